# Initial kernel scaffold; baseline (speedup 1.0000x reference)
import jax
import jax.numpy as jnp
from jax.experimental import pallas as pl


def kernel(hashes, key_vals, pred_idx, fact_scores, emb, W):
    raise NotImplementedError("write your pallas kernel here")


# speedup vs baseline: 18.9819x; 18.9819x over previous
import functools

import jax
import jax.numpy as jnp
from jax import lax
from jax.experimental import pallas as pl
from jax.experimental.pallas import tpu as pltpu
from jax.experimental.pallas import tpu_sc as plsc

E = 20000
D = 256
NB = 4
BD = D // NB
NC = 2
NS = 16
NW = NC * NS

CH = 4096
CHK = 128
QCH = 400
MSB = -2**31


def _fori(hi, body, init):
    return lax.fori_loop(jnp.int32(0), jnp.asarray(hi, jnp.int32), body, init)


def _uge(a, b):
    return (a ^ MSB) >= (b ^ MSB)


def _ge64(hi, lo, bhi, blo):
    return (hi > bhi) | ((hi == bhi) & _uge(lo, blo))


def _filter_body(lo_hbm, hi_hbm, sc_hbm, par_hbm,
                 subj_hbm, obj_hbm, scl_hbm, cnt_hbm,
                 in_lo, in_hi, in_sc, b_subj, b_obj, b_scl, par_v, cnt_v, sem):
    nch = lo_hbm.shape[1]
    cap = b_subj.shape[0]
    wid = (lax.axis_index("s") * NC + lax.axis_index("c")).astype(jnp.int32)

    pltpu.sync_copy(par_hbm, par_v)
    pv = par_v[...]
    blo_lo, blo_hi, bhi_lo, bhi_hi = pv[0], pv[1], pv[2], pv[3]

    zi = jnp.zeros((16,), jnp.int32)
    zf = jnp.zeros((16,), jnp.float32)

    def zero_body(i, _):
        b_subj[pl.ds(i * 16, 16)] = zi
        b_obj[pl.ds(i * 16, 16)] = zi
        b_scl[pl.ds(i * 16, 16)] = zf
        return 0
    _fori(cap // 16, zero_body, 0)

    inv_e = jnp.float32(1.0 / E)

    def vec_body(g, off):
        lo_v = in_lo[pl.ds(g * 16, 16)]
        hi_v = in_hi[pl.ds(g * 16, 16)]
        sc_v = in_sc[pl.ds(g * 16, 16)]
        match = (_ge64(hi_v, lo_v, blo_hi, blo_lo)
                 & ~_ge64(hi_v, lo_v, bhi_hi, bhi_lo))
        rem = lo_v - blo_lo
        q0 = (rem.astype(jnp.float32) * inv_e).astype(jnp.int32)
        r0 = rem - q0 * E
        one, zero = jnp.int32(1), jnp.int32(0)
        q1 = (q0 + jnp.where(r0 >= E, one, zero)
              - jnp.where(r0 < 0, one, zero))
        r1 = rem - q1 * E
        plsc.store_compressed(b_subj.at[pl.ds(off, 16)], q1, mask=match)
        plsc.store_compressed(b_obj.at[pl.ds(off, 16)], r1, mask=match)
        plsc.store_compressed(b_scl.at[pl.ds(off, 16)], sc_v, mask=match)
        return off + plsc.all_reduce_population_count(match)[0]

    def chunk_body(ci, off):
        d1 = pltpu.async_copy(lo_hbm.at[wid, ci], in_lo, sem)
        d2 = pltpu.async_copy(hi_hbm.at[wid, ci], in_hi, sem)
        d3 = pltpu.async_copy(sc_hbm.at[wid, ci], in_sc, sem)
        d1.wait()
        d2.wait()
        d3.wait()
        return _fori(CH // 16, vec_body, off)

    count = _fori(nch, chunk_body, jnp.int32(0))

    iota = lax.iota(jnp.int32, 16)
    cnt_v[...] = jnp.where(iota == 0, count, 0)
    pltpu.sync_copy(cnt_v, cnt_hbm.at[wid])
    pltpu.sync_copy(b_subj, subj_hbm.at[wid])
    pltpu.sync_copy(b_obj, obj_hbm.at[wid])
    pltpu.sync_copy(b_scl, scl_hbm.at[wid])


def _accum_body(subj_hbm, obj_hbm, scl_hbm, cnt_hbm, embw_hbm, zero_hbm,
                keys_hbm, out_hbm,
                shared, subj_v, idx_v, sc_v, rows_v, cnts_v, kidx, qrows,
                sem):
    c = lax.axis_index("c").astype(jnp.int32)
    s = lax.axis_index("s").astype(jnp.int32)
    stripe = (E // (NS * 8)) * 8
    rest = E - NS * stripe
    nchq = keys_hbm.shape[0]
    my_n = ((jnp.int32(nchq - 1) - s) >> 4) + 1

    pltpu.sync_copy(cnt_hbm, cnts_v)

    for slot in range(NB // NC):
        block = c * (NB // NC) + slot

        soff = pl.multiple_of(s * stripe, 8)
        pltpu.sync_copy(zero_hbm.at[pl.ds(0, stripe)],
                        shared.at[pl.ds(soff, stripe)])
        @pl.when(s == 0)
        def _():
            pltpu.sync_copy(zero_hbm.at[pl.ds(0, rest)],
                            shared.at[pl.ds(NS * stripe, rest)])
        plsc.subcore_barrier()

        for k in range(2):
            lid = s * 2 + k
            cv = cnts_v[pl.ds(lid * 16, 16)]
            ntr = (cv[0] + CHK - 1) >> 7

            def chunk_body(j, _, lid=lid, block=block):
                d1 = pltpu.async_copy(subj_hbm.at[lid, j], subj_v, sem)
                d2 = pltpu.async_copy(obj_hbm.at[lid, j], idx_v, sem)
                d3 = pltpu.async_copy(scl_hbm.at[lid, j], sc_v, sem)
                d1.wait()
                d2.wait()
                d3.wait()
                for g in range(CHK // 16):
                    ov = idx_v[pl.ds(g * 16, 16)]
                    idx_v[pl.ds(g * 16, 16)] = ov * NB + block
                pltpu.async_copy(embw_hbm.at[idx_v], rows_v, sem).wait()
                for g in range(CHK // 16):
                    sv = sc_v[pl.ds(g * 16, 16)]
                    for j2 in range(16):
                        f = g * 16 + j2
                        scl = sv[j2]
                        for k2 in range(BD // 16):
                            seg = rows_v[f, pl.ds(k2 * 16, 16)]
                            rows_v[f, pl.ds(k2 * 16, 16)] = seg * scl
                pltpu.sync_copy(rows_v, shared.at[subj_v], add=True)
                return 0

            _fori(ntr, chunk_body, 0)

        plsc.subcore_barrier()

        for cc in range(NC):
            @pl.when(c == cc)
            def _(cc=cc, slot=slot):
                col0 = (cc * (NB // NC) + slot) * BD

                def q_body(i, _):
                    t = s + i * NS
                    pltpu.sync_copy(keys_hbm.at[t], kidx)
                    pltpu.async_copy(shared.at[kidx], qrows, sem).wait()
                    roff = pl.multiple_of(t * QCH, 8)
                    pltpu.sync_copy(
                        qrows,
                        out_hbm.at[pl.ds(roff, QCH), pl.ds(col0, BD)])
                    return 0
                _fori(my_n, q_body, 0)
        plsc.subcore_barrier()


def _mm_body(a_ref, w_ref, o_ref):
    o_ref[...] = jnp.dot(a_ref[...], w_ref[...],
                         preferred_element_type=jnp.float32)


def _embw(emb, W):
    rb = 2000
    return pl.pallas_call(
        _mm_body,
        grid=(E // rb,),
        in_specs=[pl.BlockSpec((rb, D), lambda i: (i, jnp.int32(0))),
                  pl.BlockSpec((D, D),
                               lambda i: (jnp.int32(0), jnp.int32(0))),],
        out_specs=pl.BlockSpec((rb, D), lambda i: (i, jnp.int32(0))),
        out_shape=jax.ShapeDtypeStruct((E, D), jnp.float32),
    )(emb, W)


def _mesh():
    return plsc.VectorSubcoreMesh(core_axis_name="c", subcore_axis_name="s")


def _run_filter(lo3, hi3, sc3, par, cap):
    k1 = pl.kernel(
        _filter_body,
        out_type=(jax.ShapeDtypeStruct((NW, cap), jnp.int32),
                  jax.ShapeDtypeStruct((NW, cap), jnp.int32),
                  jax.ShapeDtypeStruct((NW, cap), jnp.float32),
                  jax.ShapeDtypeStruct((NW, 16), jnp.int32)),
        mesh=_mesh(),
        compiler_params=pltpu.CompilerParams(needs_layout_passes=False, use_tc_tiling_on_sc=False),
        scratch_types=[
            pltpu.VMEM((CH,), jnp.int32),
            pltpu.VMEM((CH,), jnp.int32),
            pltpu.VMEM((CH,), jnp.float32),
            pltpu.VMEM((cap,), jnp.int32),
            pltpu.VMEM((cap,), jnp.int32),
            pltpu.VMEM((cap,), jnp.float32),
            pltpu.VMEM((16,), jnp.int32),
            pltpu.VMEM((16,), jnp.int32),
            pltpu.SemaphoreType.DMA,
        ],
    )
    return k1(lo3, hi3, sc3, par)


def _run_accum(subj_l, obj_l, scl_l, cnts, embw, keys2, q, cap):
    np_ = cap // CHK
    k2 = pl.kernel(
        _accum_body,
        out_type=jax.ShapeDtypeStruct((q, D), jnp.float32),
        mesh=_mesh(),
        compiler_params=pltpu.CompilerParams(needs_layout_passes=False, use_tc_tiling_on_sc=False),
        scratch_types=[
            pltpu.VMEM_SHARED((E, BD), jnp.float32),
            pltpu.VMEM((CHK,), jnp.int32),
            pltpu.VMEM((CHK,), jnp.int32),
            pltpu.VMEM((CHK,), jnp.float32),
            pltpu.VMEM((CHK, BD), jnp.float32),
            pltpu.VMEM((NW * 16,), jnp.int32),
            pltpu.VMEM((QCH,), jnp.int32),
            pltpu.VMEM((QCH, BD), jnp.float32),
            pltpu.SemaphoreType.DMA,
        ],
    )
    zero_blk = jnp.zeros(((E // (NS * 8)) * 8, BD), jnp.float32)
    return k2(subj_l.reshape(NW, np_, CHK), obj_l.reshape(NW, np_, CHK),
              scl_l.reshape(NW, np_, CHK), cnts.reshape(NW * 16), embw,
              zero_blk, keys2)


def kernel(hashes, key_vals, pred_idx, fact_scores, emb, W):
    n = hashes.shape[0]
    q = key_vals.shape[0]

    perw = ((n + NW - 1) // NW + CH - 1) // CH * CH
    npad = perw * NW
    nch = perw // CH
    cap = perw + CHK

    h32 = lax.bitcast_convert_type(hashes, jnp.int32)
    pad = jnp.broadcast_to(jnp.array([0, -1], jnp.int32), (npad - n, 2))
    h32 = jnp.concatenate([h32, pad], axis=0)
    lo3 = h32[:, 0].reshape(NW, nch, CH)
    hi3 = h32[:, 1].reshape(NW, nch, CH)
    sc3 = jnp.concatenate(
        [fact_scores, jnp.zeros((npad - n,), jnp.float32)]).reshape(
            NW, nch, CH)

    p64 = jnp.asarray(pred_idx, jnp.int64)
    blo = p64 * (E * E)
    bhi = blo + (E * E)
    par = jnp.stack([
        (blo & 0xFFFFFFFF).astype(jnp.int32), (blo >> 32).astype(jnp.int32),
        (bhi & 0xFFFFFFFF).astype(jnp.int32), (bhi >> 32).astype(jnp.int32),
    ])
    par = jnp.concatenate([par, jnp.zeros((12,), jnp.int32)])

    subj_l, obj_l, scl_l, cnts = _run_filter(lo3, hi3, sc3, par, cap)
    embw = _embw(emb, W).reshape(E * NB, BD)
    keys2 = key_vals.astype(jnp.int32).reshape(q // QCH, QCH)
    return _run_accum(subj_l, obj_l, scl_l, cnts, embw, keys2, q, cap)
